# Initial kernel scaffold; baseline (speedup 1.0000x reference)
import jax
import jax.numpy as jnp
from jax.experimental import pallas as pl

_DIM = 256
_NCH = 2
_NEIGHBORS = 50
_N = 512
_SCALE = _DIM ** (-0.5)


def _edge_kernel(x_ref, wq0_ref, wk0_ref, wq1_ref, wk1_ref, out_ref):
    x = x_ref[0]
    attns = []
    for wq_ref, wk_ref in ((wq0_ref, wk0_ref), (wq1_ref, wk1_ref)):
        q = jnp.dot(x, wq_ref[...], preferred_element_type=jnp.float32)
        k = jnp.dot(x, wk_ref[...], preferred_element_type=jnp.float32)
        logits = jax.lax.dot_general(
            q, k, (((1,), (1,)), ((), ())),
            preferred_element_type=jnp.float32) * _SCALE
        m = jnp.max(logits, axis=-1, keepdims=True)
        e = jnp.exp(logits - m)
        s = jnp.sum(e, axis=-1, keepdims=True)
        attns.append(e / s)

    sum_edge = attns[0] + attns[1]
    bits = jax.lax.bitcast_convert_type(sum_edge, jnp.int32)
    rowmax = jnp.max(bits, axis=-1, keepdims=True)
    lo0 = jnp.zeros_like(rowmax)
    hi0 = rowmax + 1

    def body(_, carry):
        lo, hi = carry
        mid = lo + jax.lax.shift_right_logical(hi - lo, 1)
        ge = (bits >= mid).astype(jnp.float32)
        cnt = jnp.sum(ge, axis=-1, keepdims=True)
        pred = cnt >= float(_NEIGHBORS)
        return jnp.where(pred, mid, lo), jnp.where(pred, hi, mid)

    lo, _ = jax.lax.fori_loop(0, 31, body, (lo0, hi0))

    row_ids = jax.lax.broadcasted_iota(jnp.int32, (_N, _N), 0)
    col_ids = jax.lax.broadcasted_iota(jnp.int32, (_N, _N), 1)
    mask = (bits >= lo) | (row_ids == col_ids)

    for c in range(_NCH):
        edge = jnp.where(mask, attns[c], 0.0)
        nr = edge / (jnp.sum(edge, axis=-1, keepdims=True) + 1e-6)
        nc = nr / (jnp.sum(nr, axis=0, keepdims=True) + 1e-6)
        out_ref[0, c] = jax.lax.dot_general(
            nr, nc, (((1,), (1,)), ((), ())),
            preferred_element_type=jnp.float32)


def kernel(x, W):
    B, N, D = x.shape
    Wq0 = W[0 * D:1 * D].T
    Wq1 = W[1 * D:2 * D].T
    Wk0 = W[2 * D:3 * D].T
    Wk1 = W[3 * D:4 * D].T
    wspec = pl.BlockSpec((D, D), lambda b: (0, 0))
    return pl.pallas_call(
        _edge_kernel,
        grid=(B,),
        in_specs=[
            pl.BlockSpec((1, N, D), lambda b: (b, 0, 0)),
            wspec, wspec, wspec, wspec,
        ],
        out_specs=pl.BlockSpec((1, _NCH, N, N), lambda b: (b, 0, 0, 0)),
        out_shape=jax.ShapeDtypeStruct((B, _NCH, N, N), jnp.float32),
    )(x, Wq0, Wk0, Wq1, Wk1)


# speedup vs baseline: 6.2951x; 6.2951x over previous
import jax
import jax.numpy as jnp
from jax.experimental import pallas as pl

_DIM = 256
_NCH = 2
_NEIGHBORS = 50
_N = 512
_SCALE = _DIM ** (-0.5)


def _edge_kernel(x_ref, wq0_ref, wk0_ref, wq1_ref, wk1_ref, out_ref):
    x = x_ref[0]
    attns = []
    for wq_ref, wk_ref in ((wq0_ref, wk0_ref), (wq1_ref, wk1_ref)):
        q = jnp.dot(x, wq_ref[...], preferred_element_type=jnp.float32)
        k = jnp.dot(x, wk_ref[...], preferred_element_type=jnp.float32)
        logits = jax.lax.dot_general(
            q, k, (((1,), (1,)), ((), ())),
            preferred_element_type=jnp.float32) * _SCALE
        m = jnp.max(logits, axis=-1, keepdims=True)
        e = jnp.exp(logits - m)
        s = jnp.sum(e, axis=-1, keepdims=True)
        attns.append(e / s)

    sum_edge = attns[0] + attns[1]
    bits = jax.lax.bitcast_convert_type(sum_edge, jnp.int32)
    rowmax = jnp.max(bits, axis=-1, keepdims=True)

    def make_body(kth):
        def body(_, carry):
            lo, hi = carry
            mid = lo + jax.lax.shift_right_logical(hi - lo, 1)
            ge = (bits >= mid).astype(jnp.float32)
            cnt = jnp.sum(ge, axis=-1, keepdims=True)
            pred = cnt >= float(kth)
            return jnp.where(pred, mid, lo), jnp.where(pred, hi, mid)
        return body

    zeros = jnp.zeros_like(rowmax)
    v50, _ = jax.lax.fori_loop(0, 31, make_body(_NEIGHBORS), (zeros, rowmax + 1))
    v51, _ = jax.lax.fori_loop(0, 31, make_body(_NEIGHBORS + 1), (zeros, v50))
    thr = v51 + jax.lax.shift_right_logical(v50 - v51 + 1, 1)

    row_ids = jax.lax.broadcasted_iota(jnp.int32, (_N, _N), 0)
    col_ids = jax.lax.broadcasted_iota(jnp.int32, (_N, _N), 1)
    mask = (bits >= thr) | (row_ids == col_ids)

    for c in range(_NCH):
        edge = jnp.where(mask, attns[c], 0.0)
        nr = edge / (jnp.sum(edge, axis=-1, keepdims=True) + 1e-6)
        nc = nr / (jnp.sum(nr, axis=0, keepdims=True) + 1e-6)
        out_ref[0, c] = jax.lax.dot_general(
            nr, nc, (((1,), (1,)), ((), ())),
            preferred_element_type=jnp.float32)


def kernel(x, W):
    B, N, D = x.shape
    Wq0 = W[0 * D:1 * D].T
    Wq1 = W[1 * D:2 * D].T
    Wk0 = W[2 * D:3 * D].T
    Wk1 = W[3 * D:4 * D].T
    wspec = pl.BlockSpec((D, D), lambda b: (0, 0))
    return pl.pallas_call(
        _edge_kernel,
        grid=(B,),
        in_specs=[
            pl.BlockSpec((1, N, D), lambda b: (b, 0, 0)),
            wspec, wspec, wspec, wspec,
        ],
        out_specs=pl.BlockSpec((1, _NCH, N, N), lambda b: (b, 0, 0, 0)),
        out_shape=jax.ShapeDtypeStruct((B, _NCH, N, N), jnp.float32),
    )(x, Wq0, Wk0, Wq1, Wk1)


# speedup vs baseline: 10.2393x; 1.6265x over previous
import jax
import jax.numpy as jnp
from jax.experimental import pallas as pl

_DIM = 256
_NCH = 2
_NEIGHBORS = 50
_N = 512
_SCALE = _DIM ** (-0.5)


def _edge_kernel(x_ref, wq0_ref, wk0_ref, wq1_ref, wk1_ref, out_ref):
    x = x_ref[0]
    attns = []
    for wq_ref, wk_ref in ((wq0_ref, wk0_ref), (wq1_ref, wk1_ref)):
        q = jnp.dot(x, wq_ref[...], preferred_element_type=jnp.float32)
        k = jnp.dot(x, wk_ref[...], preferred_element_type=jnp.float32)
        logits = jax.lax.dot_general(
            q, k, (((1,), (1,)), ((), ())),
            preferred_element_type=jnp.float32) * _SCALE
        m = jnp.max(logits, axis=-1, keepdims=True)
        e = jnp.exp(logits - m)
        s = jnp.sum(e, axis=-1, keepdims=True)
        attns.append(e / s)

    sum_edge = attns[0] + attns[1]
    bits = jax.lax.bitcast_convert_type(sum_edge, jnp.int32)
    rowmax = jnp.max(bits, axis=-1, keepdims=True)

    def body(_, carry):
        lo, hi = carry
        mid = lo + jax.lax.shift_right_logical(hi - lo, 1)
        ge = (bits >= mid).astype(jnp.float32)
        cnt = jnp.sum(ge, axis=-1, keepdims=True)
        pred = cnt >= float(_NEIGHBORS)
        return jnp.where(pred, mid, lo), jnp.where(pred, hi, mid)

    zeros = jnp.zeros_like(rowmax)
    v50, _ = jax.lax.fori_loop(0, 31, body, (zeros, rowmax + 1))
    v51 = jnp.max(jnp.where(bits < v50, bits, 0), axis=-1, keepdims=True)
    thr = v51 + jax.lax.shift_right_logical(v50 - v51 + 1, 1)

    row_ids = jax.lax.broadcasted_iota(jnp.int32, (_N, _N), 0)
    col_ids = jax.lax.broadcasted_iota(jnp.int32, (_N, _N), 1)
    mask = (bits >= thr) | (row_ids == col_ids)

    for c in range(_NCH):
        edge = jnp.where(mask, attns[c], 0.0)
        nr = edge / (jnp.sum(edge, axis=-1, keepdims=True) + 1e-6)
        nc = nr / (jnp.sum(nr, axis=0, keepdims=True) + 1e-6)
        out_ref[0, c] = jax.lax.dot_general(
            nr, nc, (((1,), (1,)), ((), ())),
            preferred_element_type=jnp.float32)


def kernel(x, W):
    B, N, D = x.shape
    Wq0 = W[0 * D:1 * D].T
    Wq1 = W[1 * D:2 * D].T
    Wk0 = W[2 * D:3 * D].T
    Wk1 = W[3 * D:4 * D].T
    wspec = pl.BlockSpec((D, D), lambda b: (0, 0))
    return pl.pallas_call(
        _edge_kernel,
        grid=(B,),
        in_specs=[
            pl.BlockSpec((1, N, D), lambda b: (b, 0, 0)),
            wspec, wspec, wspec, wspec,
        ],
        out_specs=pl.BlockSpec((1, _NCH, N, N), lambda b: (b, 0, 0, 0)),
        out_shape=jax.ShapeDtypeStruct((B, _NCH, N, N), jnp.float32),
    )(x, Wq0, Wk0, Wq1, Wk1)


# speedup vs baseline: 11.2962x; 1.1032x over previous
import jax
import jax.numpy as jnp
from jax.experimental import pallas as pl

_DIM = 256
_NCH = 2
_NEIGHBORS = 50
_N = 512
_B = 4
_SCALE = _DIM ** (-0.5)


def _edge_kernel(x_ref, wq0_ref, wk0_ref, wq1_ref, wk1_ref, out_ref):
    x = x_ref[...].reshape(_B * _N, _DIM)
    attns = []
    for wq_ref, wk_ref in ((wq0_ref, wk0_ref), (wq1_ref, wk1_ref)):
        q = jnp.dot(x, wq_ref[...], preferred_element_type=jnp.float32)
        k = jnp.dot(x, wk_ref[...], preferred_element_type=jnp.float32)
        logits = jnp.concatenate([
            jax.lax.dot_general(
                q[b * _N:(b + 1) * _N], k[b * _N:(b + 1) * _N],
                (((1,), (1,)), ((), ())),
                preferred_element_type=jnp.float32)
            for b in range(_B)
        ], axis=0) * _SCALE
        m = jnp.max(logits, axis=-1, keepdims=True)
        e = jnp.exp(logits - m)
        s = jnp.sum(e, axis=-1, keepdims=True)
        attns.append(e / s)

    sum_edge = attns[0] + attns[1]
    bits = jax.lax.bitcast_convert_type(sum_edge, jnp.int32)
    rowmax = jnp.max(bits, axis=-1, keepdims=True)

    def body(_, carry):
        lo, hi = carry
        mid = lo + jax.lax.shift_right_logical(hi - lo, 1)
        ge = (bits >= mid).astype(jnp.float32)
        cnt = jnp.sum(ge, axis=-1, keepdims=True)
        pred = cnt >= float(_NEIGHBORS)
        return jnp.where(pred, mid, lo), jnp.where(pred, hi, mid)

    zeros = jnp.zeros_like(rowmax)
    v50, _ = jax.lax.fori_loop(0, 31, body, (zeros, rowmax + 1))
    v51 = jnp.max(jnp.where(bits < v50, bits, 0), axis=-1, keepdims=True)
    thr = v51 + jax.lax.shift_right_logical(v50 - v51 + 1, 1)

    row_ids = jax.lax.broadcasted_iota(jnp.int32, (_B * _N, _N), 0)
    col_ids = jax.lax.broadcasted_iota(jnp.int32, (_B * _N, _N), 1)
    diag = (row_ids % _N) == col_ids
    mask = (bits >= thr) | diag

    for c in range(_NCH):
        edge = jnp.where(mask, attns[c], 0.0)
        nr = edge / (jnp.sum(edge, axis=-1, keepdims=True) + 1e-6)
        for b in range(_B):
            nr_b = nr[b * _N:(b + 1) * _N]
            nc_b = nr_b / (jnp.sum(nr_b, axis=0, keepdims=True) + 1e-6)
            out_ref[b, c] = jax.lax.dot_general(
                nr_b, nc_b, (((1,), (1,)), ((), ())),
                preferred_element_type=jnp.float32)


def kernel(x, W):
    B, N, D = x.shape
    Wq0 = W[0 * D:1 * D].T
    Wq1 = W[1 * D:2 * D].T
    Wk0 = W[2 * D:3 * D].T
    Wk1 = W[3 * D:4 * D].T
    return pl.pallas_call(
        _edge_kernel,
        in_specs=[
            pl.BlockSpec((B, N, D), lambda: (0, 0, 0)),
            pl.BlockSpec((D, D), lambda: (0, 0)),
            pl.BlockSpec((D, D), lambda: (0, 0)),
            pl.BlockSpec((D, D), lambda: (0, 0)),
            pl.BlockSpec((D, D), lambda: (0, 0)),
        ],
        out_specs=pl.BlockSpec((B, _NCH, N, N), lambda: (0, 0, 0, 0)),
        out_shape=jax.ShapeDtypeStruct((B, _NCH, N, N), jnp.float32),
    )(x, Wq0, Wk0, Wq1, Wk1)
